# baseline (speedup 1.0000x reference)
import functools

import jax
import jax.numpy as jnp
from jax import lax
from jax.experimental import pallas as pl
from jax.experimental.pallas import tpu as pltpu
from jax.experimental.pallas import tpu_sc as plsc

VOCAB = 1000000
DIM = 16
BATCH = 16384

NC = 2
NS = 16
NW = NC * NS
LANES = 16

B_PER_W = BATCH // NW
CHUNK = 128
N_CHUNKS = B_PER_W // CHUNK
N_GROUPS = B_PER_W // LANES


def _sc_body(u_hbm, v_hbm, table_hbm, out_hbm,
             idx_u, idx_v, u_rows, v_rows, out_loc, sem):
    wid = lax.axis_index("s") * NC + lax.axis_index("c")
    base = wid * B_PER_W

    for c in range(N_CHUNKS):
        pltpu.sync_copy(u_hbm.at[pl.ds(base + c * CHUNK, CHUNK)], idx_u.at[c])
        pltpu.sync_copy(v_hbm.at[pl.ds(base + c * CHUNK, CHUNK)], idx_v.at[c])

    copies = []
    for c in range(N_CHUNKS):
        copies.append(pltpu.async_copy(
            table_hbm.at[idx_u.at[c]], u_rows.at[pl.ds(c * CHUNK, CHUNK)], sem))
        copies.append(pltpu.async_copy(
            table_hbm.at[idx_v.at[c]], v_rows.at[pl.ds(c * CHUNK, CHUNK)], sem))
    for cp in copies:
        cp.wait()

    lane = lax.iota(jnp.int32, LANES)

    def group(g, _):
        acc = jnp.zeros((LANES,), jnp.float32)
        for j in range(LANES):
            i = g * LANES + j
            s = jnp.sum(u_rows[i, :] * v_rows[i, :])
            acc = jnp.where(lane == j, jnp.broadcast_to(s, (LANES,)), acc)
        out_loc[pl.ds(g * LANES, LANES)] = 1.0 / (1.0 + jnp.exp(-acc))
        return _

    lax.fori_loop(0, N_GROUPS, group, None)

    pltpu.sync_copy(out_loc, out_hbm.at[pl.ds(base, B_PER_W)])


@jax.jit
def kernel(u, v, embed):
    mesh = plsc.VectorSubcoreMesh(
        core_axis_name="c", subcore_axis_name="s",
        num_cores=NC, num_subcores=NS,
    )
    k = pl.kernel(
        _sc_body,
        out_type=jax.ShapeDtypeStruct((BATCH,), jnp.float32),
        mesh=mesh,
        scratch_types=[
            pltpu.VMEM((N_CHUNKS, CHUNK), jnp.int32),
            pltpu.VMEM((N_CHUNKS, CHUNK), jnp.int32),
            pltpu.VMEM((B_PER_W, DIM), jnp.float32),
            pltpu.VMEM((B_PER_W, DIM), jnp.float32),
            pltpu.VMEM((B_PER_W,), jnp.float32),
            pltpu.SemaphoreType.DMA,
        ],
        compiler_params=pltpu.CompilerParams(
            needs_layout_passes=False, use_tc_tiling_on_sc=False),
    )
    return k(u.astype(jnp.int32), v.astype(jnp.int32), embed)
